# baseline (speedup 1.0000x reference)
import jax
import jax.numpy as jnp
from jax import lax
from jax.experimental import pallas as pl
from jax.experimental.pallas import tpu as pltpu
from jax.experimental.pallas import tpu_sc as plsc

NUM_CORES = 2
NUM_SUBCORES = 16
NW = NUM_CORES * NUM_SUBCORES
LANES = 16
BATCH = 16384
EMBED = 32
TABLE_N = 1_000_000
B_PER_W = BATCH // NW
ICHUNK = 128
N_ICHUNK = B_PER_W // ICHUNK
N_GRP = B_PER_W // LANES


def _bpr_body(uids_hbm, iids_hbm, ut_hbm, it_hbm, ub_hbm, ib_hbm, out_hbm,
              uidx, iidx, uvals, ivals, ubv, ibv, outv, sem_g, sem_b):
    c = lax.axis_index("c")
    s = lax.axis_index("s")
    wid = s * NUM_CORES + c
    base = wid * B_PER_W

    pltpu.sync_copy(uids_hbm.at[pl.ds(base, B_PER_W)], uidx)
    pltpu.sync_copy(iids_hbm.at[pl.ds(base, B_PER_W)], iidx)

    bias_copies = []
    for j in range(N_ICHUNK):
        isl = pl.ds(j * ICHUNK, ICHUNK)
        bias_copies.append(
            pltpu.async_copy(ub_hbm.at[uidx.at[isl]], ubv.at[isl], sem_b))
        bias_copies.append(
            pltpu.async_copy(ib_hbm.at[iidx.at[isl]], ibv.at[isl], sem_b))

    copies = []
    for d in range(EMBED):
        usrc = ut_hbm.at[d // 8, d % 8]
        isrc = it_hbm.at[d // 8, d % 8]
        for j in range(N_ICHUNK):
            isl = pl.ds(j * ICHUNK, ICHUNK)
            copies.append(pltpu.async_copy(
                usrc.at[uidx.at[isl]], uvals.at[d, isl], sem_g))
            copies.append(pltpu.async_copy(
                isrc.at[iidx.at[isl]], ivals.at[d, isl], sem_g))
    for cp in bias_copies:
        cp.wait()
    for cp in copies:
        cp.wait()

    def group_body(g, carry):
        goff = pl.multiple_of(g * LANES, LANES)
        acc = ubv[pl.ds(goff, LANES)] + ibv[pl.ds(goff, LANES)]
        for d in range(EMBED):
            acc = acc + uvals[d, pl.ds(goff, LANES)] * ivals[d, pl.ds(goff, LANES)]
        outv[pl.ds(goff, LANES)] = acc
        return carry

    lax.fori_loop(0, N_GRP, group_body, 0)

    pltpu.sync_copy(outv, out_hbm.at[pl.ds(base, B_PER_W)])


def kernel(user_ids, item_ids, user_embedding, item_embedding, user_bias, item_bias):
    ut = user_embedding.T.reshape(EMBED // 8, 8, TABLE_N)
    it = item_embedding.T.reshape(EMBED // 8, 8, TABLE_N)
    ubias = user_bias.reshape(-1)
    ibias = item_bias.reshape(-1)

    mesh = plsc.VectorSubcoreMesh(
        core_axis_name="c", subcore_axis_name="s",
        num_cores=NUM_CORES, num_subcores=NUM_SUBCORES,
    )
    run = pl.kernel(
        _bpr_body,
        out_type=jax.ShapeDtypeStruct((BATCH,), jnp.float32),
        mesh=mesh,
        scratch_types=[
            pltpu.VMEM((B_PER_W,), jnp.int32),
            pltpu.VMEM((B_PER_W,), jnp.int32),
            pltpu.VMEM((EMBED, B_PER_W), jnp.float32),
            pltpu.VMEM((EMBED, B_PER_W), jnp.float32),
            pltpu.VMEM((B_PER_W,), jnp.float32),
            pltpu.VMEM((B_PER_W,), jnp.float32),
            pltpu.VMEM((B_PER_W,), jnp.float32),
            pltpu.SemaphoreType.DMA,
            pltpu.SemaphoreType.DMA,
        ],
        compiler_params=pltpu.CompilerParams(
            needs_layout_passes=False, use_tc_tiling_on_sc=False,
        ),
    )
    return run(user_ids, item_ids, ut, it, ubias, ibias)
